# baseline (speedup 1.0000x reference)
import jax
import jax.numpy as jnp
from jax import lax
from jax.experimental import pallas as pl
from jax.experimental.pallas import tpu as pltpu
from jax.experimental.pallas import tpu_sc as plsc

_N = 10000
_E = 320000
_D = 128

_CHUNK = 128
_NC = 2
_NS = 16
_NW = _NC * _NS
_EPW = _E // _NW
_NT = 80
_NTH = 40
_RPT = 632
_NPAD = _RPT * _NS
_TRASH = _NPAD - 1
_TAIL = _RPT - 4 * _CHUNK

_mesh = lambda: plsc.VectorSubcoreMesh(core_axis_name="c", subcore_axis_name="s",
                                       num_cores=_NC, num_subcores=_NS)


def _init_acc(zsrc, acc_sh, base):
    for t in range(4):
        pltpu.sync_copy(zsrc, acc_sh.at[pl.ds(base + t * _CHUNK, _CHUNK)])
    pltpu.sync_copy(zsrc.at[pl.ds(0, _TAIL)],
                    acc_sh.at[pl.ds(base + 4 * _CHUNK, _TAIL)])


def _copy_out(acc_sh, out, c, base):
    for t in range(4):
        off = base + t * _CHUNK
        pltpu.sync_copy(acc_sh.at[pl.ds(off, _CHUNK)], out.at[c, pl.ds(off, _CHUNK)])
    tail = base + 4 * _CHUNK
    pltpu.sync_copy(acc_sh.at[pl.ds(tail, _TAIL)], out.at[c, pl.ds(tail, _TAIL)])


def _feat_body(srcp_hbm, dstp_hbm, node_hbm, zeros_hbm, parts_out,
               idx_s, idx_d, rows0, rows1, acc_sh, gsem, ssem):
    c = lax.axis_index("c")
    s = lax.axis_index("s")
    wid = s * _NC + c
    bufs = (rows0, rows1)
    k = len(bufs)

    base = s * _RPT
    _init_acc(zeros_hbm, acc_sh, base)
    plsc.subcore_barrier()

    for h in range(_NT // _NTH):
        h0 = wid * _NT + h * _NTH
        pltpu.sync_copy(srcp_hbm.at[pl.ds(h0, _NTH)], idx_s)
        pltpu.sync_copy(dstp_hbm.at[pl.ds(h0, _NTH)], idx_d)

        def outer(u, _):
            gds = [pltpu.async_copy(node_hbm.at[idx_s.at[k * u + b]], bufs[b], gsem)
                   for b in range(k)]
            sds = []
            for b in range(k):
                gds[b].wait()
                sds.append(pltpu.async_copy(bufs[b], acc_sh.at[idx_d.at[k * u + b]],
                                            ssem, add=True))
            for sd in sds:
                sd.wait()
            return 0
        lax.fori_loop(0, _NTH // k, outer, 0)
    plsc.subcore_barrier()

    _copy_out(acc_sh, parts_out, c, base)


def _deg_body(dstp_hbm, ones_hbm, zeros_hbm, degp_out, idx_d, ones, deg_sh, dsem):
    c = lax.axis_index("c")
    s = lax.axis_index("s")
    wid = s * _NC + c

    pltpu.sync_copy(ones_hbm, ones)
    base = s * _RPT
    _init_acc(zeros_hbm, deg_sh, base)
    plsc.subcore_barrier()

    for h in range(_NT // _NTH):
        pltpu.sync_copy(dstp_hbm.at[pl.ds(wid * _NT + h * _NTH, _NTH)], idx_d)

        def outer(u, _):
            ds = [pltpu.async_copy(ones, deg_sh.at[idx_d.at[8 * u + k]], dsem,
                                   add=True)
                  for k in range(8)]
            for d in ds:
                d.wait()
            return 0
        lax.fori_loop(0, _NTH // 8, outer, 0)
    plsc.subcore_barrier()

    _copy_out(deg_sh, degp_out, c, base)


def _sc_aggregate(srcp, dstp, node_feats):
    feat = pl.kernel(
        _feat_body,
        out_type=jax.ShapeDtypeStruct((_NC, _NPAD, _D), jnp.float32),
        mesh=_mesh(),
        scratch_types=[
            pltpu.VMEM((_NTH, _CHUNK), jnp.int32),
            pltpu.VMEM((_NTH, _CHUNK), jnp.int32),
            pltpu.VMEM((_CHUNK, _D), jnp.float32),
            pltpu.VMEM((_CHUNK, _D), jnp.float32),
            pltpu.VMEM_SHARED((_NPAD, _D), jnp.float32),
            pltpu.SemaphoreType.DMA,
            pltpu.SemaphoreType.DMA,
        ],
    )
    deg = pl.kernel(
        _deg_body,
        out_type=jax.ShapeDtypeStruct((_NC, _NPAD, _D), jnp.float32),
        mesh=_mesh(),
        scratch_types=[
            pltpu.VMEM((_NTH, _CHUNK), jnp.int32),
            pltpu.VMEM((_CHUNK, _D), jnp.float32),
            pltpu.VMEM_SHARED((_NPAD, _D), jnp.float32),
            pltpu.SemaphoreType.DMA,
        ],
    )
    zeros128 = jnp.zeros((_CHUNK, _D), jnp.float32)
    ones128 = jnp.ones((_CHUNK, _D), jnp.float32)
    return feat(srcp, dstp, node_feats, zeros128), deg(dstp, ones128, zeros128)


def _dense_body(x_ref, p_ref, d_ref, ws_ref, wn_ref, wr_ref,
                bias_ref, bres_ref, gamma_ref, beta_ref, out_ref):
    x = x_ref[...]
    agg = p_ref[0, :_N] + p_ref[1, :_N]
    deg = d_ref[0, :_N, 0:1] + d_ref[1, :_N, 0:1]
    h_neigh = agg / jnp.maximum(deg, 1.0)
    rst = (jnp.dot(x, ws_ref[...], preferred_element_type=jnp.float32)
           + jnp.dot(h_neigh, wn_ref[...], preferred_element_type=jnp.float32)
           + bias_ref[...])
    res = jnp.maximum(
        jnp.dot(x, wr_ref[...], preferred_element_type=jnp.float32) + bres_ref[...],
        0.0)
    h = rst + res
    mean = jnp.mean(h, axis=0, keepdims=True)
    var = jnp.mean((h - mean) ** 2, axis=0, keepdims=True)
    out_ref[...] = ((h - mean) * lax.rsqrt(var + 1e-5)) * gamma_ref[...] + beta_ref[...]


@jax.jit
def kernel(node_feats, edge_index, W_self, W_neigh, bias, W_res, b_res, gamma, beta):
    src = edge_index[0].reshape(_NW, _EPW)
    dst = edge_index[1].reshape(_NW, _EPW)
    srcp = jnp.zeros((_NW, _NT * _CHUNK), jnp.int32)
    srcp = srcp.at[:, :_EPW].set(src).reshape(_NW * _NT, _CHUNK)
    dstp = jnp.full((_NW, _NT * _CHUNK), _TRASH, jnp.int32)
    dstp = dstp.at[:, :_EPW].set(dst).reshape(_NW * _NT, _CHUNK)
    parts, degp = _sc_aggregate(srcp, dstp, node_feats)
    return pl.pallas_call(
        _dense_body,
        out_shape=jax.ShapeDtypeStruct((_N, _D), jnp.float32),
    )(node_feats, parts, degp, W_self, W_neigh, W_res,
      bias.reshape(1, _D), b_res.reshape(1, _D),
      gamma.reshape(1, _D), beta.reshape(1, _D))


# speedup vs baseline: 1.0105x; 1.0105x over previous
import jax
import jax.numpy as jnp
from jax import lax
from jax.experimental import pallas as pl
from jax.experimental.pallas import tpu as pltpu
from jax.experimental.pallas import tpu_sc as plsc

_N = 10000
_E = 320000
_D = 128

_CHUNK = 128
_NC = 2
_NS = 16
_NW = _NC * _NS
_EPW = _E // _NW
_NT = 80
_NTH = 40
_RPT = 632
_NPAD = _RPT * _NS
_TAIL = _RPT - 4 * _CHUNK

_mesh = lambda: plsc.VectorSubcoreMesh(core_axis_name="c", subcore_axis_name="s",
                                       num_cores=_NC, num_subcores=_NS)


def _init_acc(zsrc, acc_sh, base):
    for t in range(4):
        pltpu.sync_copy(zsrc, acc_sh.at[pl.ds(base + t * _CHUNK, _CHUNK)])
    pltpu.sync_copy(zsrc.at[pl.ds(0, _TAIL)],
                    acc_sh.at[pl.ds(base + 4 * _CHUNK, _TAIL)])


def _copy_out(acc_sh, out, c, base):
    for t in range(4):
        off = base + t * _CHUNK
        pltpu.sync_copy(acc_sh.at[pl.ds(off, _CHUNK)], out.at[c, pl.ds(off, _CHUNK)])
    tail = base + 4 * _CHUNK
    pltpu.sync_copy(acc_sh.at[pl.ds(tail, _TAIL)], out.at[c, pl.ds(tail, _TAIL)])


def _sc_body(srcp_hbm, dstp_hbm, node_hbm, zeros_hbm, ones_hbm,
             parts_out, degp_out, idx_s, idx_d, rows0, rows1, acc_sh,
             gsem, ssem):
    c = lax.axis_index("c")
    s = lax.axis_index("s")
    wid = s * _NC + c
    bufs = (rows0, rows1)
    k = len(bufs)

    base = s * _RPT
    _init_acc(zeros_hbm, acc_sh, base)
    plsc.subcore_barrier()

    for h in range(_NT // _NTH):
        h0 = wid * _NT + h * _NTH
        pltpu.sync_copy(srcp_hbm.at[pl.ds(h0, _NTH)], idx_s)
        pltpu.sync_copy(dstp_hbm.at[pl.ds(h0, _NTH)], idx_d)

        def outer(u, _):
            gds = [pltpu.async_copy(node_hbm.at[idx_s.at[k * u + b]], bufs[b], gsem)
                   for b in range(k)]
            sds = []
            for b in range(k):
                gds[b].wait()
                sds.append(pltpu.async_copy(bufs[b], acc_sh.at[idx_d.at[k * u + b]],
                                            ssem, add=True))
            for sd in sds:
                sd.wait()
            return 0
        lax.fori_loop(0, _NTH // k, outer, 0)
    plsc.subcore_barrier()

    _copy_out(acc_sh, parts_out, c, base)

    _init_acc(zeros_hbm, acc_sh, base)
    pltpu.sync_copy(ones_hbm, rows0)
    plsc.subcore_barrier()

    for h in range(_NT // _NTH):
        pltpu.sync_copy(dstp_hbm.at[pl.ds(wid * _NT + h * _NTH, _NTH)], idx_d)

        def outer(u, _):
            ds = [pltpu.async_copy(rows0, acc_sh.at[idx_d.at[8 * u + j]], gsem,
                                   add=True)
                  for j in range(8)]
            for d in ds:
                d.wait()
            return 0
        lax.fori_loop(0, _NTH // 8, outer, 0)
    plsc.subcore_barrier()

    _copy_out(acc_sh, degp_out, c, base)


def _sc_aggregate(srcp, dstp, node_feats):
    kfn = pl.kernel(
        _sc_body,
        out_type=(jax.ShapeDtypeStruct((_NC, _NPAD, _D), jnp.float32),
                  jax.ShapeDtypeStruct((_NC, _NPAD, _D), jnp.float32)),
        mesh=_mesh(),
        scratch_types=[
            pltpu.VMEM((_NTH, _CHUNK), jnp.int32),
            pltpu.VMEM((_NTH, _CHUNK), jnp.int32),
            pltpu.VMEM((_CHUNK, _D), jnp.float32),
            pltpu.VMEM((_CHUNK, _D), jnp.float32),
            pltpu.VMEM_SHARED((_NPAD, _D), jnp.float32),
            pltpu.SemaphoreType.DMA,
            pltpu.SemaphoreType.DMA,
        ],
    )
    zeros128 = jnp.zeros((_CHUNK, _D), jnp.float32)
    ones128 = jnp.ones((_CHUNK, _D), jnp.float32)
    return kfn(srcp, dstp, node_feats, zeros128, ones128)


def _dense_body(x_ref, p_ref, d_ref, ws_ref, wn_ref, wr_ref,
                bias_ref, bres_ref, gamma_ref, beta_ref, out_ref):
    x = x_ref[...]
    agg = p_ref[0, :_N] + p_ref[1, :_N]
    deg = d_ref[0, :_N, 0:1] + d_ref[1, :_N, 0:1]
    h_neigh = agg / jnp.maximum(deg, 1.0)
    rst = (jnp.dot(x, ws_ref[...], preferred_element_type=jnp.float32)
           + jnp.dot(h_neigh, wn_ref[...], preferred_element_type=jnp.float32)
           + bias_ref[...])
    res = jnp.maximum(
        jnp.dot(x, wr_ref[...], preferred_element_type=jnp.float32) + bres_ref[...],
        0.0)
    h = rst + res
    mean = jnp.mean(h, axis=0, keepdims=True)
    var = jnp.mean((h - mean) ** 2, axis=0, keepdims=True)
    out_ref[...] = ((h - mean) * lax.rsqrt(var + 1e-5)) * gamma_ref[...] + beta_ref[...]


@jax.jit
def kernel(node_feats, edge_index, W_self, W_neigh, bias, W_res, b_res, gamma, beta):
    src = edge_index[0].reshape(_NW, _EPW)
    dst = edge_index[1].reshape(_NW, _EPW)
    srcp = jnp.zeros((_NW, _NT * _CHUNK), jnp.int32)
    srcp = srcp.at[:, :_EPW].set(src).reshape(_NW * _NT, _CHUNK)
    pad_dst = _N + jnp.arange(_NT * _CHUNK, dtype=jnp.int32) % (_NPAD - _N)
    dstp = jnp.broadcast_to(pad_dst, (_NW, _NT * _CHUNK))
    dstp = dstp.at[:, :_EPW].set(dst).reshape(_NW * _NT, _CHUNK)
    parts, degp = _sc_aggregate(srcp, dstp, node_feats)
    return pl.pallas_call(
        _dense_body,
        out_shape=jax.ShapeDtypeStruct((_N, _D), jnp.float32),
    )(node_feats, parts, degp, W_self, W_neigh, W_res,
      bias.reshape(1, _D), b_res.reshape(1, _D),
      gamma.reshape(1, _D), beta.reshape(1, _D))


# speedup vs baseline: 1.3837x; 1.3693x over previous
import jax
import jax.numpy as jnp
from jax import lax
from jax.experimental import pallas as pl
from jax.experimental.pallas import tpu as pltpu
from jax.experimental.pallas import tpu_sc as plsc

_N = 10000
_E = 320000
_D = 128

_CHUNK = 128
_NCHUNKS = _E // _CHUNK
_NC = 2
_NS = 16
_NW = _NC * _NS
_RPT = 632
_NPAD = _RPT * _NS
_TAIL = _RPT - 4 * _CHUNK

_mesh = lambda: plsc.VectorSubcoreMesh(core_axis_name="c", subcore_axis_name="s",
                                       num_cores=_NC, num_subcores=_NS)


def _init_acc(zsrc, acc_sh, base):
    for t in range(4):
        pltpu.sync_copy(zsrc, acc_sh.at[pl.ds(base + t * _CHUNK, _CHUNK)])
    pltpu.sync_copy(zsrc.at[pl.ds(0, _TAIL)],
                    acc_sh.at[pl.ds(base + 4 * _CHUNK, _TAIL)])


def _copy_out(acc_sh, out, c, base):
    for t in range(4):
        off = base + t * _CHUNK
        pltpu.sync_copy(acc_sh.at[pl.ds(off, _CHUNK)], out.at[c, pl.ds(off, _CHUNK)])
    tail = base + 4 * _CHUNK
    pltpu.sync_copy(acc_sh.at[pl.ds(tail, _TAIL)], out.at[c, pl.ds(tail, _TAIL)])


def _nloc(wid):
    return _NCHUNKS // _NW + jnp.where(wid < _NCHUNKS % _NW, 1, 0)


def _sc_body(src_hbm, dst_hbm, node_hbm, zeros_hbm, ones_hbm,
             parts_out, degp_out, src_idx, dst_idx, rows, acc_sh, sem):
    c = lax.axis_index("c")
    s = lax.axis_index("s")
    wid = s * _NC + c

    base = s * _RPT
    _init_acc(zeros_hbm, acc_sh, base)
    plsc.subcore_barrier()

    def step(t, _):
        e0 = (wid + t * _NW) * _CHUNK
        pltpu.sync_copy(src_hbm.at[pl.ds(e0, _CHUNK)], src_idx)
        pltpu.sync_copy(dst_hbm.at[pl.ds(e0, _CHUNK)], dst_idx)
        pltpu.async_copy(node_hbm.at[src_idx], rows, sem).wait()
        pltpu.sync_copy(rows, acc_sh.at[dst_idx], add=True)
        return 0
    lax.fori_loop(0, _nloc(wid), step, 0)
    plsc.subcore_barrier()

    _copy_out(acc_sh, parts_out, c, base)

    _init_acc(zeros_hbm, acc_sh, base)
    pltpu.sync_copy(ones_hbm, rows)
    plsc.subcore_barrier()

    def dstep(t, _):
        e0 = (wid + t * _NW) * _CHUNK
        pltpu.sync_copy(dst_hbm.at[pl.ds(e0, _CHUNK)], dst_idx)
        pltpu.sync_copy(rows, acc_sh.at[dst_idx], add=True)
        return 0
    lax.fori_loop(0, _nloc(wid), dstep, 0)
    plsc.subcore_barrier()

    _copy_out(acc_sh, degp_out, c, base)


def _sc_aggregate(src, dst, node_feats):
    kfn = pl.kernel(
        _sc_body,
        out_type=(jax.ShapeDtypeStruct((_NC, _NPAD, _D), jnp.float32),
                  jax.ShapeDtypeStruct((_NC, _NPAD, _D), jnp.float32)),
        mesh=_mesh(),
        scratch_types=[
            pltpu.VMEM((_CHUNK,), jnp.int32),
            pltpu.VMEM((_CHUNK,), jnp.int32),
            pltpu.VMEM((_CHUNK, _D), jnp.float32),
            pltpu.VMEM_SHARED((_NPAD, _D), jnp.float32),
            pltpu.SemaphoreType.DMA,
        ],
    )
    zeros128 = jnp.zeros((_CHUNK, _D), jnp.float32)
    ones128 = jnp.ones((_CHUNK, _D), jnp.float32)
    return kfn(src, dst, node_feats, zeros128, ones128)


def _dense_body(x_ref, p_ref, d_ref, ws_ref, wn_ref, wr_ref,
                bias_ref, bres_ref, gamma_ref, beta_ref, out_ref):
    x = x_ref[...]
    agg = p_ref[0, :_N] + p_ref[1, :_N]
    deg = d_ref[0, :_N, 0:1] + d_ref[1, :_N, 0:1]
    h_neigh = agg / jnp.maximum(deg, 1.0)
    rst = (jnp.dot(x, ws_ref[...], preferred_element_type=jnp.float32)
           + jnp.dot(h_neigh, wn_ref[...], preferred_element_type=jnp.float32)
           + bias_ref[...])
    res = jnp.maximum(
        jnp.dot(x, wr_ref[...], preferred_element_type=jnp.float32) + bres_ref[...],
        0.0)
    h = rst + res
    mean = jnp.mean(h, axis=0, keepdims=True)
    var = jnp.mean((h - mean) ** 2, axis=0, keepdims=True)
    out_ref[...] = ((h - mean) * lax.rsqrt(var + 1e-5)) * gamma_ref[...] + beta_ref[...]


@jax.jit
def kernel(node_feats, edge_index, W_self, W_neigh, bias, W_res, b_res, gamma, beta):
    parts, degp = _sc_aggregate(edge_index[0], edge_index[1], node_feats)
    return pl.pallas_call(
        _dense_body,
        out_shape=jax.ShapeDtypeStruct((_N, _D), jnp.float32),
    )(node_feats, parts, degp, W_self, W_neigh, W_res,
      bias.reshape(1, _D), b_res.reshape(1, _D),
      gamma.reshape(1, _D), beta.reshape(1, _D))


# speedup vs baseline: 1.8342x; 1.3256x over previous
import jax
import jax.numpy as jnp
from jax import lax
from jax.experimental import pallas as pl
from jax.experimental.pallas import tpu as pltpu
from jax.experimental.pallas import tpu_sc as plsc

_N = 10000
_E = 320000
_D = 128

_CHUNK = 128
_NCHUNKS = _E // _CHUNK
_NC = 2
_NS = 16
_NW = _NC * _NS
_RPT = 632
_NPAD = _RPT * _NS
_TAIL = _RPT - 4 * _CHUNK

_mesh = lambda: plsc.VectorSubcoreMesh(core_axis_name="c", subcore_axis_name="s",
                                       num_cores=_NC, num_subcores=_NS)


def _init_acc(zsrc, acc_sh, base):
    for t in range(4):
        pltpu.sync_copy(zsrc, acc_sh.at[pl.ds(base + t * _CHUNK, _CHUNK)])
    pltpu.sync_copy(zsrc.at[pl.ds(0, _TAIL)],
                    acc_sh.at[pl.ds(base + 4 * _CHUNK, _TAIL)])


def _copy_out(acc_sh, out, c, base):
    for t in range(4):
        off = base + t * _CHUNK
        pltpu.sync_copy(acc_sh.at[pl.ds(off, _CHUNK)], out.at[c, pl.ds(off, _CHUNK)])
    tail = base + 4 * _CHUNK
    pltpu.sync_copy(acc_sh.at[pl.ds(tail, _TAIL)], out.at[c, pl.ds(tail, _TAIL)])


_NPAIR = 39
_NTAILC = _NCHUNKS - 78 * _NW


def _e0(wid, t):
    return (wid + t * _NW) * _CHUNK


def _sc_body(src_hbm, dst_hbm, node_hbm, zeros_hbm, ones_hbm,
             parts_out, degp_out, sA, dA, sB, dB, rows, acc_sh,
             sem, isemA, isemB):
    c = lax.axis_index("c")
    s = lax.axis_index("s")
    wid = s * _NC + c

    base = s * _RPT
    _init_acc(zeros_hbm, acc_sh, base)
    plsc.subcore_barrier()

    pltpu.sync_copy(src_hbm.at[pl.ds(_e0(wid, 0), _CHUNK)], sA)
    pltpu.sync_copy(dst_hbm.at[pl.ds(_e0(wid, 0), _CHUNK)], dA)
    pltpu.sync_copy(src_hbm.at[pl.ds(_e0(wid, 1), _CHUNK)], sB)
    pltpu.sync_copy(dst_hbm.at[pl.ds(_e0(wid, 1), _CHUNK)], dB)

    def pair(u, _):
        t0 = 2 * u
        t1 = 2 * u + 1

        @pl.when(u > 0)
        def _():
            pltpu.make_async_copy(src_hbm.at[pl.ds(_e0(wid, t0), _CHUNK)], sA, isemA).wait()
            pltpu.make_async_copy(dst_hbm.at[pl.ds(_e0(wid, t0), _CHUNK)], dA, isemA).wait()
        pltpu.async_copy(node_hbm.at[sA], rows, sem).wait()
        pltpu.sync_copy(rows, acc_sh.at[dA], add=True)

        @pl.when(u < _NPAIR - 1)
        def _():
            pltpu.async_copy(src_hbm.at[pl.ds(_e0(wid, t0 + 2), _CHUNK)], sA, isemA)
            pltpu.async_copy(dst_hbm.at[pl.ds(_e0(wid, t0 + 2), _CHUNK)], dA, isemA)

        @pl.when(u > 0)
        def _():
            pltpu.make_async_copy(src_hbm.at[pl.ds(_e0(wid, t1), _CHUNK)], sB, isemB).wait()
            pltpu.make_async_copy(dst_hbm.at[pl.ds(_e0(wid, t1), _CHUNK)], dB, isemB).wait()
        pltpu.async_copy(node_hbm.at[sB], rows, sem).wait()
        pltpu.sync_copy(rows, acc_sh.at[dB], add=True)

        @pl.when(u < _NPAIR - 1)
        def _():
            pltpu.async_copy(src_hbm.at[pl.ds(_e0(wid, t1 + 2), _CHUNK)], sB, isemB)
            pltpu.async_copy(dst_hbm.at[pl.ds(_e0(wid, t1 + 2), _CHUNK)], dB, isemB)
        return 0
    lax.fori_loop(0, _NPAIR, pair, 0)

    @pl.when(wid < _NTAILC)
    def _():
        e0 = (78 * _NW + wid) * _CHUNK
        pltpu.sync_copy(src_hbm.at[pl.ds(e0, _CHUNK)], sA)
        pltpu.sync_copy(dst_hbm.at[pl.ds(e0, _CHUNK)], dA)
        pltpu.async_copy(node_hbm.at[sA], rows, sem).wait()
        pltpu.sync_copy(rows, acc_sh.at[dA], add=True)
    plsc.subcore_barrier()

    _copy_out(acc_sh, parts_out, c, base)

    _init_acc(zeros_hbm, acc_sh, base)
    pltpu.sync_copy(ones_hbm, rows)
    plsc.subcore_barrier()

    pltpu.sync_copy(dst_hbm.at[pl.ds(_e0(wid, 0), _CHUNK)], dA)
    pltpu.sync_copy(dst_hbm.at[pl.ds(_e0(wid, 1), _CHUNK)], dB)

    def dpair(u, _):
        t0 = 2 * u
        t1 = 2 * u + 1

        @pl.when(u > 0)
        def _():
            pltpu.make_async_copy(dst_hbm.at[pl.ds(_e0(wid, t0), _CHUNK)], dA, isemA).wait()
        pltpu.sync_copy(rows, acc_sh.at[dA], add=True)

        @pl.when(u < _NPAIR - 1)
        def _():
            pltpu.async_copy(dst_hbm.at[pl.ds(_e0(wid, t0 + 2), _CHUNK)], dA, isemA)

        @pl.when(u > 0)
        def _():
            pltpu.make_async_copy(dst_hbm.at[pl.ds(_e0(wid, t1), _CHUNK)], dB, isemB).wait()
        pltpu.sync_copy(rows, acc_sh.at[dB], add=True)

        @pl.when(u < _NPAIR - 1)
        def _():
            pltpu.async_copy(dst_hbm.at[pl.ds(_e0(wid, t1 + 2), _CHUNK)], dB, isemB)
        return 0
    lax.fori_loop(0, _NPAIR, dpair, 0)

    @pl.when(wid < _NTAILC)
    def _():
        e0 = (78 * _NW + wid) * _CHUNK
        pltpu.sync_copy(dst_hbm.at[pl.ds(e0, _CHUNK)], dA)
        pltpu.sync_copy(rows, acc_sh.at[dA], add=True)
    plsc.subcore_barrier()

    _copy_out(acc_sh, degp_out, c, base)


def _sc_aggregate(src, dst, node_feats):
    kfn = pl.kernel(
        _sc_body,
        out_type=(jax.ShapeDtypeStruct((_NC, _NPAD, _D), jnp.float32),
                  jax.ShapeDtypeStruct((_NC, _NPAD, _D), jnp.float32)),
        mesh=_mesh(),
        scratch_types=[
            pltpu.VMEM((_CHUNK,), jnp.int32),
            pltpu.VMEM((_CHUNK,), jnp.int32),
            pltpu.VMEM((_CHUNK,), jnp.int32),
            pltpu.VMEM((_CHUNK,), jnp.int32),
            pltpu.VMEM((_CHUNK, _D), jnp.float32),
            pltpu.VMEM_SHARED((_NPAD, _D), jnp.float32),
            pltpu.SemaphoreType.DMA,
            pltpu.SemaphoreType.DMA,
            pltpu.SemaphoreType.DMA,
        ],
    )
    zeros128 = jnp.zeros((_CHUNK, _D), jnp.float32)
    ones128 = jnp.ones((_CHUNK, _D), jnp.float32)
    return kfn(src, dst, node_feats, zeros128, ones128)


def _dense_body(x_ref, p_ref, d_ref, ws_ref, wn_ref, wr_ref,
                bias_ref, bres_ref, gamma_ref, beta_ref, out_ref):
    x = x_ref[...]
    agg = p_ref[0, :_N] + p_ref[1, :_N]
    deg = d_ref[0, :_N, 0:1] + d_ref[1, :_N, 0:1]
    h_neigh = agg / jnp.maximum(deg, 1.0)
    rst = (jnp.dot(x, ws_ref[...], preferred_element_type=jnp.float32)
           + jnp.dot(h_neigh, wn_ref[...], preferred_element_type=jnp.float32)
           + bias_ref[...])
    res = jnp.maximum(
        jnp.dot(x, wr_ref[...], preferred_element_type=jnp.float32) + bres_ref[...],
        0.0)
    h = rst + res
    mean = jnp.mean(h, axis=0, keepdims=True)
    var = jnp.mean((h - mean) ** 2, axis=0, keepdims=True)
    out_ref[...] = ((h - mean) * lax.rsqrt(var + 1e-5)) * gamma_ref[...] + beta_ref[...]


@jax.jit
def kernel(node_feats, edge_index, W_self, W_neigh, bias, W_res, b_res, gamma, beta):
    parts, degp = _sc_aggregate(edge_index[0], edge_index[1], node_feats)
    return pl.pallas_call(
        _dense_body,
        out_shape=jax.ShapeDtypeStruct((_N, _D), jnp.float32),
    )(node_feats, parts, degp, W_self, W_neigh, W_res,
      bias.reshape(1, _D), b_res.reshape(1, _D),
      gamma.reshape(1, _D), beta.reshape(1, _D))
